# Initial kernel scaffold; baseline (speedup 1.0000x reference)
import jax
import jax.numpy as jnp
from jax.experimental import pallas as pl


def kernel(depth_vmap, depth_nmap, live_verts, vert_normals, valid_verts):
    raise NotImplementedError("write your pallas kernel here")


# speedup vs baseline: 26.7484x; 26.7484x over previous
import functools
import math

import jax
import jax.numpy as jnp
from jax import lax
from jax.experimental import pallas as pl
from jax.experimental.pallas import tpu as pltpu

_NV = 6890
_ND = 8192
_NQ = 7168
_BQ = 512
_BK = 512

_TH2 = 0.05 * 0.05
_CT = math.cos(math.pi / 12.0)
_VIS2 = 0.5 * 0.5
_IDXM = 0x1FFF
_SENT = 0x7FFFFFFF


def _bf(x):
    return x.astype(jnp.bfloat16).astype(jnp.float32)


def _norm_rows(nx, ny, nz):
    inv = 1.0 / jnp.sqrt(nx * nx + ny * ny + nz * nz + 1e-12)
    return nx * inv, ny * inv, nz * inv


def _pass_rows(qx, qy, qz, qhx, qhy, qhz, key_cols):
    qq = qx * qx + qy * qy + qz * qz
    qbx, qby, qbz = _bf(qx), _bf(qy), _bf(qz)
    qnx, qny, qnz = _bf(qhx), _bf(qhy), _bf(qhz)
    nkc_total = None

    def kchunk(kb, carry):
        row_all, row_val = carry
        kxf, kyf, kzf, nhx, nhy, nhz, ks = key_cols(kb)
        kk = kxf * kxf + kyf * kyf + kzf * kzf
        kbx, kby, kbz = _bf(kxf), _bf(kyf), _bf(kzf)
        nbx, nby, nbz = _bf(nhx), _bf(nhy), _bf(nhz)
        cross = kbx * qbx + kby * qby + kbz * qbz
        d2 = jnp.maximum((kk - 2.0 * cross) + qq, 0.0)
        cos = nbx * qnx + nby * qny + nbz * qnz
        valid = (d2 < _TH2) & (cos > _CT)
        bits = lax.bitcast_convert_type(d2, jnp.int32)
        iot = lax.broadcasted_iota(jnp.int32, (_BK, 1), 0) + ks
        packed = (bits & jnp.int32(~_IDXM)) | iot
        pval = jnp.where(valid, packed, jnp.int32(_SENT))
        row_all = jnp.minimum(row_all, jnp.min(packed, axis=0, keepdims=True))
        row_val = jnp.minimum(row_val, jnp.min(pval, axis=0, keepdims=True))
        return row_all, row_val

    return kchunk


def _s2d_body(lvT, vnT, validq, dvm, dnm, vis_out, corr_out):
    nqb = _NQ // _BQ
    nkc = _ND // _BK

    def qblock(qb, _):
        qs = qb * _BQ
        qx = lvT[0:1, pl.ds(qs, _BQ)]
        qy = lvT[1:2, pl.ds(qs, _BQ)]
        qz = lvT[2:3, pl.ds(qs, _BQ)]
        qhx, qhy, qhz = _norm_rows(
            vnT[0:1, pl.ds(qs, _BQ)],
            vnT[1:2, pl.ds(qs, _BQ)],
            vnT[2:3, pl.ds(qs, _BQ)],
        )

        def key_cols(kb):
            ks = kb * _BK
            k = dvm[pl.ds(ks, _BK), :]
            n = dnm[pl.ds(ks, _BK), :]
            nhx, nhy, nhz = _norm_rows(n[:, 0:1], n[:, 1:2], n[:, 2:3])
            return k[:, 0:1], k[:, 1:2], k[:, 2:3], nhx, nhy, nhz, ks

        kchunk = _pass_rows(qx, qy, qz, qhx, qhy, qhz, key_cols)
        init = (jnp.full((1, _BQ), _SENT, jnp.int32),
                jnp.full((1, _BQ), _SENT, jnp.int32))
        row_all, row_val = lax.fori_loop(0, nkc, kchunk, init)
        d2min = lax.bitcast_convert_type(
            row_all & jnp.int32(~_IDXM), jnp.float32)
        visb = jnp.where(
            (d2min < _VIS2) & (validq[0:1, pl.ds(qs, _BQ)] > 0.5), 1.0, 0.0
        ).astype(jnp.float32)
        corr = jnp.where(row_val != jnp.int32(_SENT), row_val, row_all)
        vis_out[0:1, pl.ds(qs, _BQ)] = visb
        corr_out[0:1, pl.ds(qs, _BQ)] = corr & jnp.int32(_IDXM)
        return 0

    lax.fori_loop(0, nqb, qblock, 0)


def _d2s_body(dvmT, dnmT, lv, vn, visc, corr_out):
    nqb = _ND // _BQ
    nkc = _NQ // _BK

    def qblock(qb, _):
        qs = qb * _BQ
        qx = dvmT[0:1, pl.ds(qs, _BQ)]
        qy = dvmT[1:2, pl.ds(qs, _BQ)]
        qz = dvmT[2:3, pl.ds(qs, _BQ)]
        qhx, qhy, qhz = _norm_rows(
            dnmT[0:1, pl.ds(qs, _BQ)],
            dnmT[1:2, pl.ds(qs, _BQ)],
            dnmT[2:3, pl.ds(qs, _BQ)],
        )

        def key_cols(kb):
            ks = kb * _BK
            l = lv[pl.ds(ks, _BK), :]
            m = visc[pl.ds(ks, _BK), :] > 0.5
            kxf = jnp.where(m, l[:, 0:1], 1e6)
            kyf = jnp.where(m, l[:, 1:2], 1e6)
            kzf = jnp.where(m, l[:, 2:3], 1e6)
            v = vn[pl.ds(ks, _BK), :]
            nhx, nhy, nhz = _norm_rows(v[:, 0:1], v[:, 1:2], v[:, 2:3])
            return kxf, kyf, kzf, nhx, nhy, nhz, ks

        kchunk = _pass_rows(qx, qy, qz, qhx, qhy, qhz, key_cols)
        init = (jnp.full((1, _BQ), _SENT, jnp.int32),
                jnp.full((1, _BQ), _SENT, jnp.int32))
        row_all, row_val = lax.fori_loop(0, nkc, kchunk, init)
        corr = jnp.where(row_val != jnp.int32(_SENT), row_val, row_all)
        corr_out[0:1, pl.ds(qs, _BQ)] = corr & jnp.int32(_IDXM)
        return 0

    lax.fori_loop(0, nqb, qblock, 0)


def _loss_body(vis, ds2, dd2, loss_out):
    dist = jnp.sqrt(ds2[...] + 1e-12)
    s = jnp.sum(vis[...] * dist, axis=1, keepdims=True)
    v = jnp.sum(vis[...], axis=1, keepdims=True)
    t = jnp.sum(jnp.sqrt(dd2[...] + 1e-12), axis=1, keepdims=True)
    loss_out[:, :] = s / jnp.maximum(v, 1.0) + t / float(_ND)


@functools.partial(jax.jit, static_argnames=("interpret",))
def _run(depth_vmap, depth_nmap, live_verts, vert_normals, valid_verts,
         interpret=False):
    f32 = jnp.float32
    pad = _NQ - _NV
    lvp = jnp.pad(live_verts, ((0, pad), (0, 0)))
    vnp = jnp.pad(vert_normals, ((0, pad), (0, 0)))
    lvT = lvp.T
    vnT = vnp.T
    validq = jnp.pad(valid_verts, (0, pad)).reshape(1, _NQ)

    vis, corr_s2d = pl.pallas_call(
        _s2d_body,
        out_shape=[
            jax.ShapeDtypeStruct((1, _NQ), f32),
            jax.ShapeDtypeStruct((1, _NQ), jnp.int32),
        ],
        interpret=interpret,
    )(lvT, vnT, validq, depth_vmap, depth_nmap)

    visc = vis.reshape(_NQ, 1)
    corr_d2s = pl.pallas_call(
        _d2s_body,
        out_shape=jax.ShapeDtypeStruct((1, _ND), jnp.int32),
        interpret=interpret,
    )(depth_vmap.T, depth_nmap.T, lvp, vnp, visc)

    gs = depth_vmap[corr_s2d.reshape(-1)]
    dx = lvp - gs
    ds2 = jnp.sum(dx * dx, axis=1).reshape(1, _NQ)
    gd = lvp[corr_d2s.reshape(-1)]
    dy = depth_vmap - gd
    dd2 = jnp.sum(dy * dy, axis=1).reshape(1, _ND)

    loss = pl.pallas_call(
        _loss_body,
        out_shape=jax.ShapeDtypeStruct((1, 1), f32),
        interpret=interpret,
    )(vis, ds2, dd2)
    return loss.reshape(())


def kernel(depth_vmap, depth_nmap, live_verts, vert_normals, valid_verts):
    return _run(depth_vmap, depth_nmap, live_verts, vert_normals, valid_verts)
